# baseline (speedup 1.0000x reference)
import functools

import jax
import jax.numpy as jnp
from jax import lax
from jax.experimental import pallas as pl
from jax.experimental.pallas import tpu as pltpu, tpu_sc as plsc

EMB_DIM = 128
VOCAB = 48
NUM_CORES = 2
NUM_SUBCORES = 16
NUM_WORKERS = NUM_CORES * NUM_SUBCORES
CHUNK = 128
NBUF = 4
BLK = 80
SUP = BLK // NBUF


@functools.partial(jax.jit, static_argnames=("total_rows",))
def _lookup(idx_chunks, table, *, total_rows):
    chunks_total = total_rows // CHUNK
    chunks_per_worker = chunks_total // NUM_WORKERS
    nblk = chunks_per_worker // BLK
    supers_per_worker = chunks_per_worker // NBUF
    mesh = plsc.VectorSubcoreMesh(
        core_axis_name="c", subcore_axis_name="s",
        num_cores=NUM_CORES, num_subcores=NUM_SUBCORES)

    @functools.partial(
        pl.kernel,
        out_type=jax.ShapeDtypeStruct((total_rows, EMB_DIM), jnp.float32),
        mesh=mesh,
        scratch_types=[
            pltpu.VMEM((2, BLK, CHUNK), jnp.int32),
            pltpu.VMEM((NBUF, CHUNK, EMB_DIM), jnp.float32),
            pltpu.VMEM_SHARED((VOCAB, EMB_DIM), jnp.float32),
            [pltpu.SemaphoreType.DMA] * 2,
            [pltpu.SemaphoreType.DMA] * NBUF,
            [pltpu.SemaphoreType.DMA] * NBUF,
        ],
    )
    def body(idx_hbm, table_hbm, out_hbm, idx_v, rows_v, table_sh,
             sem_i, sem_g, sem_w):
        wid = lax.axis_index("s") * NUM_CORES + lax.axis_index("c")
        chunk0 = wid * chunks_per_worker

        @pl.when(lax.axis_index("s") == 0)
        def _():
            pltpu.sync_copy(table_hbm, table_sh)

        plsc.subcore_barrier()

        def start_idx(blk, slot):
            pltpu.async_copy(
                idx_hbm.at[pl.ds(chunk0 + blk * BLK, BLK)],
                idx_v.at[slot], sem_i[slot])

        def wait_idx(slot):
            pltpu.make_async_copy(
                idx_hbm.at[pl.ds(chunk0, BLK)], idx_v.at[slot],
                sem_i[slot]).wait()

        def wait_write(b):
            pltpu.make_async_copy(
                rows_v.at[b], out_hbm.at[pl.ds(chunk0, CHUNK)],
                sem_w[b]).wait()

        start_idx(0, 0)

        for blk in range(nblk):
            slot = blk % 2
            wait_idx(slot)
            if blk + 1 < nblk:
                start_idx(blk + 1, 1 - slot)

            def super_step(sj, carry, blk=blk, slot=slot):
                for b in range(NBUF):
                    if blk == 0:
                        @pl.when(sj > 0)
                        def _(b=b):
                            wait_write(b)
                    else:
                        wait_write(b)
                    pltpu.async_copy(
                        table_sh.at[idx_v.at[slot, sj * NBUF + b]],
                        rows_v.at[b], sem_g[b])
                for b in range(NBUF):
                    pltpu.make_async_copy(
                        table_sh.at[idx_v.at[slot, sj * NBUF + b]],
                        rows_v.at[b], sem_g[b]).wait()
                    row0 = (chunk0 + blk * BLK + sj * NBUF + b) * CHUNK
                    pltpu.async_copy(
                        rows_v.at[b], out_hbm.at[pl.ds(row0, CHUNK)],
                        sem_w[b])
                return carry

            lax.fori_loop(0, SUP, super_step, 0)
        for b in range(NBUF):
            wait_write(b)

    return body(idx_chunks, table)


TC_ROWS = 1024


@functools.partial(jax.jit, static_argnames=("total_rows",))
def _lookup_tc(idx3, table_hi, table_lo, *, total_rows):
    nblocks = total_rows // TC_ROWS

    def tc_body(idx_ref, hi_ref, lo_ref, out_ref):
        iota = lax.broadcasted_iota(jnp.int32, (VOCAB, TC_ROWS), 0)
        oh = jnp.equal(idx_ref[0], iota).astype(jnp.bfloat16)
        acc = lax.dot_general(oh, hi_ref[...], (((0,), (0,)), ((), ())),
                              preferred_element_type=jnp.float32)
        acc += lax.dot_general(oh, lo_ref[...], (((0,), (0,)), ((), ())),
                               preferred_element_type=jnp.float32)
        out_ref[...] = acc

    return pl.pallas_call(
        tc_body,
        grid=(nblocks,),
        in_specs=[
            pl.BlockSpec((1, 1, TC_ROWS), lambda i: (i, 0, 0)),
            pl.BlockSpec((VOCAB, EMB_DIM), lambda i: (0, 0)),
            pl.BlockSpec((VOCAB, EMB_DIM), lambda i: (0, 0)),
        ],
        out_specs=pl.BlockSpec((TC_ROWS, EMB_DIM), lambda i: (i, 0)),
        out_shape=jax.ShapeDtypeStruct((total_rows, EMB_DIM), jnp.float32),
    )(idx3, table_hi, table_lo)


def kernel(x, emb1_weight, emb2_weight):
    b, t, _ = x.shape
    total_rows = b * t
    idx = x[..., 1].astype(jnp.int32).reshape(total_rows // TC_ROWS, 1, TC_ROWS)
    hi = emb1_weight.astype(jnp.bfloat16)
    lo = (emb1_weight - hi.astype(jnp.float32)).astype(jnp.bfloat16)
    out = _lookup_tc(idx, hi, lo, total_rows=total_rows)
    return out.reshape(b, t, EMB_DIM)


# speedup vs baseline: 1.0323x; 1.0323x over previous
import functools

import jax
import jax.numpy as jnp
from jax import lax
from jax.experimental import pallas as pl
from jax.experimental.pallas import tpu as pltpu, tpu_sc as plsc

EMB_DIM = 128
VOCAB = 48
NUM_CORES = 2
NUM_SUBCORES = 16
NUM_WORKERS = NUM_CORES * NUM_SUBCORES
CHUNK = 128
NBUF = 4
BLK = 80
SUP = BLK // NBUF


@functools.partial(jax.jit, static_argnames=("total_rows",))
def _lookup(idx_chunks, table, *, total_rows):
    chunks_total = total_rows // CHUNK
    chunks_per_worker = chunks_total // NUM_WORKERS
    nblk = chunks_per_worker // BLK
    supers_per_worker = chunks_per_worker // NBUF
    mesh = plsc.VectorSubcoreMesh(
        core_axis_name="c", subcore_axis_name="s",
        num_cores=NUM_CORES, num_subcores=NUM_SUBCORES)

    @functools.partial(
        pl.kernel,
        out_type=jax.ShapeDtypeStruct((total_rows, EMB_DIM), jnp.float32),
        mesh=mesh,
        scratch_types=[
            pltpu.VMEM((2, BLK, CHUNK), jnp.int32),
            pltpu.VMEM((NBUF, CHUNK, EMB_DIM), jnp.float32),
            pltpu.VMEM_SHARED((VOCAB, EMB_DIM), jnp.float32),
            [pltpu.SemaphoreType.DMA] * 2,
            [pltpu.SemaphoreType.DMA] * NBUF,
            [pltpu.SemaphoreType.DMA] * NBUF,
        ],
    )
    def body(idx_hbm, table_hbm, out_hbm, idx_v, rows_v, table_sh,
             sem_i, sem_g, sem_w):
        wid = lax.axis_index("s") * NUM_CORES + lax.axis_index("c")
        chunk0 = wid * chunks_per_worker

        @pl.when(lax.axis_index("s") == 0)
        def _():
            pltpu.sync_copy(table_hbm, table_sh)

        plsc.subcore_barrier()

        def start_idx(blk, slot):
            pltpu.async_copy(
                idx_hbm.at[pl.ds(chunk0 + blk * BLK, BLK)],
                idx_v.at[slot], sem_i[slot])

        def wait_idx(slot):
            pltpu.make_async_copy(
                idx_hbm.at[pl.ds(chunk0, BLK)], idx_v.at[slot],
                sem_i[slot]).wait()

        def wait_write(b):
            pltpu.make_async_copy(
                rows_v.at[b], out_hbm.at[pl.ds(chunk0, CHUNK)],
                sem_w[b]).wait()

        start_idx(0, 0)

        for blk in range(nblk):
            slot = blk % 2
            wait_idx(slot)
            if blk + 1 < nblk:
                start_idx(blk + 1, 1 - slot)

            def super_step(sj, carry, blk=blk, slot=slot):
                for b in range(NBUF):
                    if blk == 0:
                        @pl.when(sj > 0)
                        def _(b=b):
                            wait_write(b)
                    else:
                        wait_write(b)
                    pltpu.async_copy(
                        table_sh.at[idx_v.at[slot, sj * NBUF + b]],
                        rows_v.at[b], sem_g[b])
                for b in range(NBUF):
                    pltpu.make_async_copy(
                        table_sh.at[idx_v.at[slot, sj * NBUF + b]],
                        rows_v.at[b], sem_g[b]).wait()
                    row0 = (chunk0 + blk * BLK + sj * NBUF + b) * CHUNK
                    pltpu.async_copy(
                        rows_v.at[b], out_hbm.at[pl.ds(row0, CHUNK)],
                        sem_w[b])
                return carry

            lax.fori_loop(0, SUP, super_step, 0)
        for b in range(NBUF):
            wait_write(b)

    return body(idx_chunks, table)


TC_ROWS = 1024


@functools.partial(jax.jit, static_argnames=("total_rows",))
def _lookup_tc(idx3, table_hi, table_lo, *, total_rows):
    nblocks = total_rows // TC_ROWS

    def tc_body(idx_ref, hilo_ref, out_ref):
        iota = lax.broadcasted_iota(jnp.int32, (2 * VOCAB, TC_ROWS), 0)
        oh = jnp.equal(idx_ref[0], iota % VOCAB).astype(jnp.bfloat16)
        out_ref[...] = lax.dot_general(
            oh, hilo_ref[...], (((0,), (0,)), ((), ())),
            preferred_element_type=jnp.float32)

    return pl.pallas_call(
        tc_body,
        grid=(nblocks,),
        in_specs=[
            pl.BlockSpec((1, 1, TC_ROWS), lambda i: (i, 0, 0)),
            pl.BlockSpec((2 * VOCAB, EMB_DIM), lambda i: (0, 0)),
        ],
        out_specs=pl.BlockSpec((TC_ROWS, EMB_DIM), lambda i: (i, 0)),
        out_shape=jax.ShapeDtypeStruct((total_rows, EMB_DIM), jnp.float32),
    )(idx3, jnp.concatenate([table_hi, table_lo], axis=0))


def kernel(x, emb1_weight, emb2_weight):
    b, t, _ = x.shape
    total_rows = b * t
    idx = x[..., 1].astype(jnp.int32).reshape(total_rows // TC_ROWS, 1, TC_ROWS)
    hi = emb1_weight.astype(jnp.bfloat16)
    lo = (emb1_weight - hi.astype(jnp.float32)).astype(jnp.bfloat16)
    out = _lookup_tc(idx, hi, lo, total_rows=total_rows)
    return out.reshape(b, t, EMB_DIM)


# speedup vs baseline: 3.0920x; 2.9954x over previous
import functools

import jax
import jax.numpy as jnp
from jax import lax
from jax.experimental import pallas as pl
from jax.experimental.pallas import tpu as pltpu, tpu_sc as plsc

EMB_DIM = 128
VOCAB = 48
NUM_CORES = 2
NUM_SUBCORES = 16
NUM_WORKERS = NUM_CORES * NUM_SUBCORES
CHUNK = 128
NBUF = 4
BLK = 80
SUP = BLK // NBUF


@functools.partial(jax.jit, static_argnames=("total_rows",))
def _lookup(idx_chunks, table, *, total_rows):
    chunks_total = total_rows // CHUNK
    chunks_per_worker = chunks_total // NUM_WORKERS
    nblk = chunks_per_worker // BLK
    mesh = plsc.VectorSubcoreMesh(
        core_axis_name="c", subcore_axis_name="s",
        num_cores=NUM_CORES, num_subcores=NUM_SUBCORES)

    @functools.partial(
        pl.kernel,
        out_type=jax.ShapeDtypeStruct((total_rows, EMB_DIM), jnp.float32),
        mesh=mesh,
        scratch_types=[
            pltpu.VMEM((2, BLK, CHUNK), jnp.int32),
            pltpu.VMEM((NBUF, CHUNK, EMB_DIM), jnp.float32),
            pltpu.VMEM_SHARED((VOCAB, EMB_DIM), jnp.float32),
            [pltpu.SemaphoreType.DMA] * 2,
            [pltpu.SemaphoreType.DMA] * NBUF,
            [pltpu.SemaphoreType.DMA] * NBUF,
        ],
    )
    def body(idx_hbm, table_hbm, out_hbm, idx_v, rows_v, table_sh,
             sem_i, sem_g, sem_w):
        wid = lax.axis_index("s") * NUM_CORES + lax.axis_index("c")
        chunk0 = wid * chunks_per_worker

        @pl.when(lax.axis_index("s") == 0)
        def _():
            pltpu.sync_copy(table_hbm, table_sh)

        plsc.subcore_barrier()

        def start_idx(blk, slot):
            pltpu.async_copy(
                idx_hbm.at[pl.ds(chunk0 + blk * BLK, BLK)],
                idx_v.at[slot], sem_i[slot])

        def wait_idx(slot):
            pltpu.make_async_copy(
                idx_hbm.at[pl.ds(chunk0, BLK)], idx_v.at[slot],
                sem_i[slot]).wait()

        def wait_write(b):
            pltpu.make_async_copy(
                rows_v.at[b], out_hbm.at[pl.ds(chunk0, CHUNK)],
                sem_w[b]).wait()

        start_idx(0, 0)

        for blk in range(nblk):
            slot = blk % 2
            wait_idx(slot)
            if blk + 1 < nblk:
                start_idx(blk + 1, 1 - slot)

            def super_step(sj, carry, blk=blk, slot=slot):
                for b in range(NBUF):
                    if blk == 0:
                        @pl.when(sj > 0)
                        def _(b=b):
                            wait_write(b)
                    else:
                        wait_write(b)
                    pltpu.async_copy(
                        table_sh.at[idx_v.at[slot, sj * NBUF + b]],
                        rows_v.at[b], sem_g[b])
                for b in range(NBUF):
                    pltpu.make_async_copy(
                        table_sh.at[idx_v.at[slot, sj * NBUF + b]],
                        rows_v.at[b], sem_g[b]).wait()
                    row0 = (chunk0 + blk * BLK + sj * NBUF + b) * CHUNK
                    pltpu.async_copy(
                        rows_v.at[b], out_hbm.at[pl.ds(row0, CHUNK)],
                        sem_w[b])
                return carry

            lax.fori_loop(0, SUP, super_step, 0)
        for b in range(NBUF):
            wait_write(b)

    return body(idx_chunks, table)


def kernel(x, emb1_weight, emb2_weight):
    b, t, _ = x.shape
    total_rows = b * t
    idx = x[..., 1].astype(jnp.int32).reshape(total_rows // CHUNK, CHUNK)
    out = _lookup(idx, emb1_weight, total_rows=total_rows)
    return out.reshape(b, t, EMB_DIM)
